# baseline (speedup 1.0000x reference)
import functools
import jax
import jax.numpy as jnp
from jax import lax
from jax.experimental import pallas as pl
from jax.experimental.pallas import tpu as pltpu
from jax.experimental.pallas import tpu_sc as plsc

D = 64
SCALE = 8.0
G = 128
GPC = 2
C = G * GPC
NC = 2
NS = 16
NW = NC * NS


def _body(nb, na, x_hbm, t_hbm, out_hbm,
          idx_v, rows_v, tbuf, isem, gsem, osem):
    wid = lax.axis_index("s") * NC + lax.axis_index("c")
    tiles_per_b = na // C
    per_w = (nb * tiles_per_b) // NW
    fvec = [lax.iota(jnp.int32, 16) + j * 16 for j in range(D // 16)]

    def tile_coords(t):
        tile = t * NW + wid
        return tile // tiles_per_b, tile % tiles_per_b

    def start_idx(t, buf):
        b, ac = tile_coords(t)
        pltpu.async_copy(x_hbm.at[b, pl.ds(ac * GPC, GPC)],
                         idx_v.at[buf], isem)

    def start_gather(t, buf):
        pltpu.make_async_copy(x_hbm.at[0, pl.ds(0, GPC)],
                              idx_v.at[buf], isem).wait()
        for j in range(GPC):
            pltpu.async_copy(
                t_hbm.at[idx_v.at[buf, j]],
                rows_v.at[buf, pl.ds(j * G, G)],
                gsem,
            )

    start_idx(0, 0)
    start_gather(0, 0)
    start_idx(1, 1)

    def tile_body(t, carry):
        buf = lax.rem(t, 2)
        b, ac = tile_coords(t)

        @pl.when(t + 1 < per_w)
        def _():
            start_gather(t + 1, 1 - buf)

        pltpu.make_async_copy(t_hbm.at[pl.ds(0, C)],
                              rows_v.at[buf], gsem).wait()

        @pl.when(t + 2 < per_w)
        def _():
            start_idx(t + 2, buf)

        @pl.when(t >= 2)
        def _():
            pltpu.make_async_copy(t_hbm.at[pl.ds(0, C)],
                                  rows_v.at[buf], osem).wait()

        def row_body(k, carry2):
            r = rows_v.at[buf, k]
            kv = jnp.full((16,), 0, jnp.int32) + k
            for j in range(D // 16):
                v = r[pl.ds(j * 16, 16)] * SCALE
                plsc.store_scatter(tbuf.at[buf], [fvec[j], kv], v)
            return carry2

        lax.fori_loop(0, C, row_body, 0, unroll=4)

        pltpu.async_copy(tbuf.at[buf],
                         out_hbm.at[b, pl.ds(0, D), pl.ds(ac * C, C)],
                         osem)
        return carry

    lax.fori_loop(0, per_w, tile_body, 0)
    pltpu.make_async_copy(t_hbm.at[pl.ds(0, C)], rows_v.at[0], osem).wait()
    pltpu.make_async_copy(t_hbm.at[pl.ds(0, C)], rows_v.at[1], osem).wait()


@functools.partial(jax.jit, static_argnames=("nb", "na"))
def _sc_lookup(xr, tlin, nb, na):
    mesh = plsc.VectorSubcoreMesh(core_axis_name="c", subcore_axis_name="s")
    k = pl.kernel(
        functools.partial(_body, nb, na),
        mesh=mesh,
        compiler_params=pltpu.CompilerParams(
            use_tc_tiling_on_sc=False, needs_layout_passes=False
        ),
        out_type=jax.ShapeDtypeStruct((nb, D, na), jnp.float32),
        scratch_types=[
            pltpu.VMEM((2, GPC, G), jnp.int32),
            pltpu.VMEM((2, C, D), jnp.float32),
            pltpu.VMEM((2, D, C), jnp.float32),
            pltpu.SemaphoreType.DMA,
            pltpu.SemaphoreType.DMA,
            pltpu.SemaphoreType.DMA,
        ],
    )
    return k(xr, tlin)


def kernel(x, table):
    vocab = table.shape[0]
    na, nb = x.shape
    t2 = jax.lax.optimization_barrier(jnp.reshape(table, (vocab // 2, 2 * D)))
    tlin = jnp.reshape(t2, (vocab, D))
    xr = jnp.reshape(x.T, (nb, na // G, G))
    outp = _sc_lookup(xr, tlin, nb, na)
    return outp.transpose(2, 0, 1)


# speedup vs baseline: 1.1655x; 1.1655x over previous
import functools
import jax
import jax.numpy as jnp
from jax import lax
from jax.experimental import pallas as pl
from jax.experimental.pallas import tpu as pltpu
from jax.experimental.pallas import tpu_sc as plsc

D = 64
DP = 128
SCALE = 8.0
G = 128
GPC = 2
C = G * GPC
NC = 2
NS = 16
NW = NC * NS


def _body(nb, na, x_hbm, t_hbm, out_hbm,
          idx_v, rows_v, tbuf, isem, gsem, osem):
    wid = lax.axis_index("s") * NC + lax.axis_index("c")
    tiles_per_b = na // C
    per_w = (nb * tiles_per_b) // NW
    fvec = [lax.iota(jnp.int32, 16) + j * 16 for j in range(D // 16)]

    def tile_coords(t):
        tile = t * NW + wid
        return tile // tiles_per_b, tile % tiles_per_b

    def start_idx(t, buf):
        b, ac = tile_coords(t)
        pltpu.async_copy(x_hbm.at[pl.ds(b * na + ac * C, C)],
                         idx_v.at[buf], isem)

    def start_gather(t, buf):
        pltpu.make_async_copy(x_hbm.at[pl.ds(0, C)],
                              idx_v.at[buf], isem).wait()
        for j in range(GPC):
            pltpu.async_copy(
                t_hbm.at[idx_v.at[buf, pl.ds(j * G, G)]],
                rows_v.at[buf, pl.ds(j * G, G)],
                gsem,
            )

    start_idx(0, 0)
    start_gather(0, 0)
    start_idx(1, 1)

    def tile_body(t, carry):
        buf = lax.rem(t, 2)
        b, ac = tile_coords(t)

        @pl.when(t + 1 < per_w)
        def _():
            start_gather(t + 1, 1 - buf)

        pltpu.make_async_copy(t_hbm.at[pl.ds(0, C)],
                              rows_v.at[buf], gsem).wait()

        @pl.when(t + 2 < per_w)
        def _():
            start_idx(t + 2, buf)

        @pl.when(t >= 2)
        def _():
            pltpu.make_async_copy(
                out_hbm.at[0, pl.ds(0, D), pl.ds(0, C)],
                tbuf.at[buf], osem).wait()

        def row_body(k, kv):
            r = rows_v.at[buf, k]
            for j in range(D // 16):
                v = r[pl.ds(j * 16, 16)] * SCALE
                plsc.store_scatter(tbuf.at[buf], [fvec[j], kv], v)
            return kv + 1

        lax.fori_loop(0, C, row_body,
                      jnp.zeros((16,), jnp.int32), unroll=4)

        pltpu.async_copy(tbuf.at[buf],
                         out_hbm.at[b, pl.ds(0, D), pl.ds(ac * C, C)],
                         osem)
        return carry

    lax.fori_loop(0, per_w, tile_body, 0)
    pltpu.make_async_copy(out_hbm.at[0, pl.ds(0, D), pl.ds(0, C)],
                          tbuf.at[0], osem).wait()
    pltpu.make_async_copy(out_hbm.at[0, pl.ds(0, D), pl.ds(0, C)],
                          tbuf.at[1], osem).wait()


@functools.partial(jax.jit, static_argnames=("nb", "na"))
def _sc_lookup(xf, tpad, nb, na):
    mesh = plsc.VectorSubcoreMesh(core_axis_name="c", subcore_axis_name="s")
    k = pl.kernel(
        functools.partial(_body, nb, na),
        mesh=mesh,
        compiler_params=pltpu.CompilerParams(needs_layout_passes=False),
        out_type=jax.ShapeDtypeStruct((nb, D, na), jnp.float32),
        scratch_types=[
            pltpu.VMEM((2, C), jnp.int32),
            pltpu.VMEM((2, C, DP), jnp.float32),
            pltpu.VMEM((2, D, C), jnp.float32),
            pltpu.SemaphoreType.DMA,
            pltpu.SemaphoreType.DMA,
            pltpu.SemaphoreType.DMA,
        ],
    )
    return k(xf, tpad)


def kernel(x, table):
    na, nb = x.shape
    tpad = jnp.pad(table, ((0, 0), (0, DP - D)))
    xf = jnp.reshape(x.T, (nb * na,))
    outp = _sc_lookup(xf, tpad, nb, na)
    return outp.transpose(2, 0, 1)


# speedup vs baseline: 1.9327x; 1.6583x over previous
import functools
import jax
import jax.numpy as jnp
from jax import lax
from jax.experimental import pallas as pl
from jax.experimental.pallas import tpu as pltpu
from jax.experimental.pallas import tpu_sc as plsc

D = 64
DP = 128
SCALE = 8.0
G = 128
GPC = 2
C = G * GPC
NC = 2
NS = 16
NW = NC * NS
TB = 2048
SLABS = 4


def _t1_body(in_ref, out_ref):
    out_ref[:, 0:D] = in_ref[...].T * SCALE


def _table_rows(tT):
    v = tT.shape[1]
    return pl.pallas_call(
        _t1_body,
        grid=(pl.cdiv(v, TB),),
        in_specs=[pl.BlockSpec((D, TB), lambda i: (0, i))],
        out_specs=pl.BlockSpec((TB, DP), lambda i: (i, 0)),
        out_shape=jax.ShapeDtypeStruct((v, DP), jnp.float32),
    )(tT)


def _t2_body(in_ref, out_ref):
    out_ref[0] = in_ref[:, 0:D].T


def _t2_acc_body(in_ref, prev_ref, out_ref):
    out_ref[0] = in_ref[:, 0:D].T


def _to_feature_major(interm, nbs, na, nb, s, out_prev=None):
    out_map = lambda i: (i + s * nbs, 0, 0)
    if out_prev is None:
        return pl.pallas_call(
            _t2_body,
            grid=(nbs,),
            in_specs=[pl.BlockSpec((na, DP), lambda i: (i, 0))],
            out_specs=pl.BlockSpec((1, D, na), out_map),
            out_shape=jax.ShapeDtypeStruct((nb, D, na), jnp.float32),
        )(interm)
    return pl.pallas_call(
        _t2_acc_body,
        grid=(nbs,),
        in_specs=[
            pl.BlockSpec((na, DP), lambda i: (i, 0)),
            pl.BlockSpec((1, 8, 128), lambda i: (0, 0, 0)),
        ],
        out_specs=pl.BlockSpec((1, D, na), out_map),
        out_shape=jax.ShapeDtypeStruct((nb, D, na), jnp.float32),
        input_output_aliases={1: 0},
    )(interm, out_prev)


def _gather_body(n, x_hbm, t_hbm, out_hbm, idx_v, rows_v, isem, gsem, osem):
    wid = lax.axis_index("s") * NC + lax.axis_index("c")
    per_w = n // (NW * C)

    def start_idx(t, buf):
        n0 = (t * NW + wid) * C
        pltpu.async_copy(x_hbm.at[pl.ds(n0, C)], idx_v.at[buf], isem)

    def start_gather(t, buf):
        pltpu.make_async_copy(x_hbm.at[pl.ds(0, C)],
                              idx_v.at[buf], isem).wait()
        for j in range(GPC):
            pltpu.async_copy(
                t_hbm.at[idx_v.at[buf, pl.ds(j * G, G)]],
                rows_v.at[buf, pl.ds(j * G, G)],
                gsem,
            )

    def drain_out(buf):
        pltpu.make_async_copy(
            t_hbm.at[pl.ds(0, C)],
            rows_v.at[buf], osem).wait()

    start_idx(0, 0)
    start_gather(0, 0)
    start_idx(1, 1)

    def tile_body(t, carry):
        buf = lax.rem(t, 2)
        n0 = (t * NW + wid) * C

        @pl.when(t >= 1)
        def _():
            drain_out(1 - buf)

        @pl.when(t + 1 < per_w)
        def _():
            start_gather(t + 1, 1 - buf)

        pltpu.make_async_copy(t_hbm.at[pl.ds(0, C)],
                              rows_v.at[buf], gsem).wait()

        @pl.when(t + 2 < per_w)
        def _():
            start_idx(t + 2, buf)

        pltpu.async_copy(rows_v.at[buf],
                         out_hbm.at[pl.ds(n0, C)], osem)
        return carry

    lax.fori_loop(0, per_w, tile_body, 0)
    drain_out(lax.rem(per_w - 1, 2))


@functools.partial(jax.jit, static_argnames=("n",))
def _sc_gather(xf, trows, n):
    mesh = plsc.VectorSubcoreMesh(core_axis_name="c", subcore_axis_name="s")
    k = pl.kernel(
        functools.partial(_gather_body, n),
        mesh=mesh,
        compiler_params=pltpu.CompilerParams(needs_layout_passes=False),
        out_type=jax.ShapeDtypeStruct((n, DP), jnp.float32),
        scratch_types=[
            pltpu.VMEM((2, C), jnp.int32),
            pltpu.VMEM((2, C, DP), jnp.float32),
            pltpu.SemaphoreType.DMA,
            pltpu.SemaphoreType.DMA,
            pltpu.SemaphoreType.DMA,
        ],
    )
    return k(xf, trows)


def kernel(x, table):
    na, nb = x.shape
    n = na * nb
    trows = _table_rows(table.T)
    xf = jnp.reshape(x.T, (n,))
    ns = n // SLABS
    nbs = nb // SLABS
    outp = None
    for s in range(SLABS):
        interm = _sc_gather(xf[s * ns:(s + 1) * ns], trows, ns)
        outp = _to_feature_major(interm, nbs, na, nb, s, outp)
    return outp.transpose(2, 0, 1)


# speedup vs baseline: 1.9616x; 1.0149x over previous
import functools
import jax
import jax.numpy as jnp
from jax import lax
from jax.experimental import pallas as pl
from jax.experimental.pallas import tpu as pltpu
from jax.experimental.pallas import tpu_sc as plsc

D = 64
DP = 128
SCALE = 8.0
G = 128
GPC = 2
C = G * GPC
NC = 2
NS = 16
NW = NC * NS
TB = 2048
SLABS = 8


def _t1_body(in_ref, out_ref):
    out_ref[:, 0:D] = in_ref[...].T * SCALE


def _table_rows(tT):
    v = tT.shape[1]
    return pl.pallas_call(
        _t1_body,
        grid=(pl.cdiv(v, TB),),
        in_specs=[pl.BlockSpec((D, TB), lambda i: (0, i))],
        out_specs=pl.BlockSpec((TB, DP), lambda i: (i, 0)),
        out_shape=jax.ShapeDtypeStruct((v, DP), jnp.float32),
    )(tT)


def _t2_body(in_ref, out_ref):
    out_ref[0] = in_ref[:, 0:D].T


def _t2_acc_body(in_ref, prev_ref, out_ref):
    out_ref[0] = in_ref[:, 0:D].T


def _to_feature_major(interm, nbs, na, nb, s, out_prev=None):
    out_map = lambda i: (i + s * nbs, 0, 0)
    if out_prev is None:
        return pl.pallas_call(
            _t2_body,
            grid=(nbs,),
            in_specs=[pl.BlockSpec((na, DP), lambda i: (i, 0))],
            out_specs=pl.BlockSpec((1, D, na), out_map),
            out_shape=jax.ShapeDtypeStruct((nb, D, na), jnp.float32),
        )(interm)
    return pl.pallas_call(
        _t2_acc_body,
        grid=(nbs,),
        in_specs=[
            pl.BlockSpec((na, DP), lambda i: (i, 0)),
            pl.BlockSpec((1, 8, 128), lambda i: (0, 0, 0)),
        ],
        out_specs=pl.BlockSpec((1, D, na), out_map),
        out_shape=jax.ShapeDtypeStruct((nb, D, na), jnp.float32),
        input_output_aliases={1: 0},
    )(interm, out_prev)


def _gather_body(n, x_hbm, t_hbm, out_hbm, idx_v, rows_v, isem, gsem, osem):
    wid = lax.axis_index("s") * NC + lax.axis_index("c")
    per_w = n // (NW * C)

    def start_idx(t, buf):
        n0 = (t * NW + wid) * C
        pltpu.async_copy(x_hbm.at[pl.ds(n0, C)], idx_v.at[buf], isem)

    def start_gather(t, buf):
        pltpu.make_async_copy(x_hbm.at[pl.ds(0, C)],
                              idx_v.at[buf], isem).wait()
        for j in range(GPC):
            pltpu.async_copy(
                t_hbm.at[idx_v.at[buf, pl.ds(j * G, G)]],
                rows_v.at[buf, pl.ds(j * G, G)],
                gsem,
            )

    def drain_out(buf):
        pltpu.make_async_copy(
            t_hbm.at[pl.ds(0, C)],
            rows_v.at[buf], osem).wait()

    start_idx(0, 0)
    start_gather(0, 0)
    start_idx(1, 1)

    def tile_body(t, carry):
        buf = lax.rem(t, 2)
        n0 = (t * NW + wid) * C

        @pl.when(t >= 1)
        def _():
            drain_out(1 - buf)

        @pl.when(t + 1 < per_w)
        def _():
            start_gather(t + 1, 1 - buf)

        pltpu.make_async_copy(t_hbm.at[pl.ds(0, C)],
                              rows_v.at[buf], gsem).wait()

        @pl.when(t + 2 < per_w)
        def _():
            start_idx(t + 2, buf)

        pltpu.async_copy(rows_v.at[buf],
                         out_hbm.at[pl.ds(n0, C)], osem)
        return carry

    lax.fori_loop(0, per_w, tile_body, 0)
    drain_out(lax.rem(per_w - 1, 2))


@functools.partial(jax.jit, static_argnames=("n",))
def _sc_gather(xf, trows, n):
    mesh = plsc.VectorSubcoreMesh(core_axis_name="c", subcore_axis_name="s")
    k = pl.kernel(
        functools.partial(_gather_body, n),
        mesh=mesh,
        compiler_params=pltpu.CompilerParams(needs_layout_passes=False),
        out_type=jax.ShapeDtypeStruct((n, DP), jnp.float32),
        scratch_types=[
            pltpu.VMEM((2, C), jnp.int32),
            pltpu.VMEM((2, C, DP), jnp.float32),
            pltpu.SemaphoreType.DMA,
            pltpu.SemaphoreType.DMA,
            pltpu.SemaphoreType.DMA,
        ],
    )
    return k(xf, trows)


def kernel(x, table):
    na, nb = x.shape
    n = na * nb
    trows = _table_rows(table.T)
    xf = jnp.reshape(x.T, (n,))
    ns = n // SLABS
    nbs = nb // SLABS
    outp = None
    for s in range(SLABS):
        interm = _sc_gather(xf[s * ns:(s + 1) * ns], trows, ns)
        outp = _to_feature_major(interm, nbs, na, nb, s, outp)
    return outp.transpose(2, 0, 1)
